# Initial kernel scaffold; baseline (speedup 1.0000x reference)
import jax
import jax.numpy as jnp
from jax.experimental import pallas as pl


def kernel(x, edge_index, edge_types, node_types, W):
    raise NotImplementedError("write your pallas kernel here")


# speedup vs baseline: 2.7605x; 2.7605x over previous
import functools

import jax
import jax.numpy as jnp
from jax import lax
from jax.experimental import pallas as pl
from jax.experimental.pallas import tpu as pltpu
from jax.experimental.pallas import tpu_sc as plsc


def _table_body(x_ref, nt_ref, wcat_ref, tcat_ref, out_ref):
    acc = jnp.dot(x_ref[...], wcat_ref[...],
                  preferred_element_type=jnp.float32,
                  precision=lax.Precision.HIGHEST)
    nt = nt_ref[...]
    for t in range(4):
        mask = jnp.where(nt == t, 1.0, 0.0)
        acc = acc + mask * tcat_ref[t:t + 1, :]
    out_ref[...] = acc


def _make_sc_edge_kernel(n_tab_words, n_edges):
    nc, ns = 2, 16
    nw = nc * ns
    epw = n_edges // nw
    ch = 2000
    assert n_edges % nw == 0 and epw % ch == 0 and ch % 16 == 0

    mesh = plsc.VectorSubcoreMesh(core_axis_name="c", subcore_axis_name="s",
                                  num_cores=nc, num_subcores=ns)

    @functools.partial(
        pl.kernel,
        out_type=jax.ShapeDtypeStruct((n_edges * 4,), jnp.float32),
        mesh=mesh,
        compiler_params=pltpu.CompilerParams(needs_layout_passes=False),
        scratch_types=[
            pltpu.VMEM((n_tab_words,), jnp.float32),
            pltpu.VMEM((ch,), jnp.int32),
            pltpu.VMEM((ch,), jnp.int32),
            pltpu.VMEM((ch * 4,), jnp.float32),
        ],
    )
    def sc_edge_kernel(tab_hbm, src_hbm, dst_hbm, out_hbm,
                       tab_v, src_v, dst_v, out_v):
        w = lax.axis_index("s") * nc + lax.axis_index("c")
        pltpu.sync_copy(tab_hbm, tab_v)
        ebase = w * epw

        def chunk(ci, _):
            base = ebase + ci * ch
            pltpu.sync_copy(src_hbm.at[pl.ds(base, ch)], src_v)
            pltpu.sync_copy(dst_hbm.at[pl.ds(base, ch)], dst_v)

            def group(g, carry):
                s = src_v[pl.ds(g * 16, 16)]
                d = dst_v[pl.ds(g * 16, 16)]
                s8 = s * 8
                d8 = d * 8 + 4
                pos = (lax.iota(jnp.int32, 16) + g * 16) * 4
                for c in range(4):
                    a = plsc.load_gather(tab_v, [s8 + c])
                    b = plsc.load_gather(tab_v, [d8 + c])
                    v = a + b
                    e = jnp.exp(-2.0 * jnp.abs(v))
                    r = (1.0 - e) / (1.0 + e)
                    r = jnp.where(v < 0.0, -r, r)
                    plsc.store_scatter(out_v, [pos + c], r)
                return carry

            lax.fori_loop(0, ch // 16, group, 0)
            pltpu.sync_copy(out_v, out_hbm.at[pl.ds(base * 4, ch * 4)])
            return 0

        lax.fori_loop(0, epw // ch, chunk, 0)

    return sc_edge_kernel


def kernel(x, edge_index, edge_types, node_types, W):
    n, d = x.shape
    e = edge_index.shape[1]
    wcat = jnp.concatenate([W[:d], W[d:2 * d]], axis=1)
    tcat = jnp.concatenate([W[2 * d:2 * d + 4],
                            W[2 * d + 4:2 * d + 8]], axis=1)

    table = pl.pallas_call(
        _table_body,
        out_shape=jax.ShapeDtypeStruct((n, 8), jnp.float32),
    )(x, node_types.reshape(n, 1), wcat, tcat)

    tab_flat = table.reshape(-1)
    out_flat = _make_sc_edge_kernel(tab_flat.shape[0], e)(
        tab_flat, edge_index[0], edge_index[1])
    return out_flat.reshape(e, 2, 2)


# speedup vs baseline: 2.9534x; 1.0699x over previous
import functools

import jax
import jax.numpy as jnp
from jax import lax
from jax.experimental import pallas as pl
from jax.experimental.pallas import tpu as pltpu
from jax.experimental.pallas import tpu_sc as plsc


def _table_body(x_ref, nt_ref, wcat_ref, tcat_ref, out_ref):
    acc = jnp.dot(x_ref[...], wcat_ref[...],
                  preferred_element_type=jnp.float32,
                  precision=lax.Precision.HIGHEST)
    nt = nt_ref[...]
    for t in range(4):
        mask = jnp.where(nt == t, 1.0, 0.0)
        acc = acc + mask * tcat_ref[t:t + 1, :]
    out_ref[...] = acc


def _make_sc_edge_kernel(n_tab_words, n_rows):
    nc, ns = 2, 16
    nw = nc * ns
    rpw = n_rows // nw
    rc = 64
    assert n_rows % nw == 0 and rpw % rc == 0 and rpw % 8 == 0
    starts = [k * rc for k in range(rpw // rc)]
    ec = rc * 32

    mesh = plsc.VectorSubcoreMesh(core_axis_name="c", subcore_axis_name="s",
                                  num_cores=nc, num_subcores=ns)

    @functools.partial(
        pl.kernel,
        out_type=jax.ShapeDtypeStruct((n_rows, 128), jnp.float32),
        mesh=mesh,
        compiler_params=pltpu.CompilerParams(needs_layout_passes=False),
        scratch_types=[
            pltpu.VMEM((n_tab_words,), jnp.float32),
            pltpu.VMEM((ec,), jnp.int32),
            pltpu.VMEM((ec,), jnp.int32),
            pltpu.VMEM((rc, 128), jnp.float32),
        ],
    )
    def sc_edge_kernel(tab_hbm, src_hbm, dst_hbm, out_hbm,
                       tab_v, src_v, dst_v, out_v):
        w = lax.axis_index("s") * nc + lax.axis_index("c")
        pltpu.sync_copy(tab_hbm, tab_v)
        row0 = w * rpw

        for start in starts:
            row = row0 + start
            ebase = row * 32
            pltpu.sync_copy(src_hbm.at[pl.ds(ebase, ec)], src_v)
            pltpu.sync_copy(dst_hbm.at[pl.ds(ebase, ec)], dst_v)

            def group(g, carry):
                s = src_v[pl.ds(g * 16, 16)]
                d = dst_v[pl.ds(g * 16, 16)]
                s8 = s * 8
                d8 = d * 8 + 4
                p0 = (lax.iota(jnp.int32, 16) + g * 16) * 4
                for c in range(4):
                    a = plsc.load_gather(tab_v, [s8 + c])
                    b = plsc.load_gather(tab_v, [d8 + c])
                    v = a + b
                    e = jnp.exp(-2.0 * jnp.abs(v))
                    r = (1.0 - e) / (1.0 + e)
                    r = jnp.where(v < 0.0, -r, r)
                    p = p0 + c
                    plsc.store_scatter(out_v, [p >> 7, p & 127], r)
                return carry

            lax.fori_loop(0, ec // 16, group, 0)
            pltpu.sync_copy(out_v, out_hbm.at[pl.ds(row, rc)])

    return sc_edge_kernel


def kernel(x, edge_index, edge_types, node_types, W):
    n, d = x.shape
    e = edge_index.shape[1]
    wcat = jnp.concatenate([W[:d], W[d:2 * d]], axis=1)
    tcat = jnp.concatenate([W[2 * d:2 * d + 4],
                            W[2 * d + 4:2 * d + 8]], axis=1)

    table = pl.pallas_call(
        _table_body,
        out_shape=jax.ShapeDtypeStruct((n, 8), jnp.float32),
    )(x, node_types.reshape(n, 1), wcat, tcat)

    align = 32 * 64 * 32
    e_pad = -(-e // align) * align
    pad = e_pad - e
    src = edge_index[0]
    dst = edge_index[1]
    if pad:
        zeros = jnp.zeros((pad,), jnp.int32)
        src = jnp.concatenate([src, zeros])
        dst = jnp.concatenate([dst, zeros])

    tab_flat = table.reshape(-1)
    out2d = _make_sc_edge_kernel(tab_flat.shape[0], e_pad * 4 // 128)(
        tab_flat, src, dst)
    return out2d.reshape(e_pad, 2, 2)[:e]


# speedup vs baseline: 23.5950x; 7.9891x over previous
import functools

import jax
import jax.numpy as jnp
from jax import lax
from jax.experimental import pallas as pl
from jax.experimental.pallas import tpu as pltpu
from jax.experimental.pallas import tpu_sc as plsc


def _table_body(x_ref, nt_ref, wcat_ref, tcat_ref, out_ref):
    acc = jnp.dot(x_ref[...], wcat_ref[...],
                  preferred_element_type=jnp.float32,
                  precision=lax.Precision.HIGHEST)
    nt = nt_ref[...]
    for t in range(4):
        mask = jnp.where(nt == t, 1.0, 0.0)
        acc = acc + mask * tcat_ref[t:t + 1, :]
    out_ref[...] = acc


def _make_sc_edge_kernel(n_tab_words, n_edges):
    nc, ns = 2, 16
    nw = nc * ns
    assert n_edges % 128 == 0
    n_blk = n_edges // 128
    blk_lo = n_blk // nw
    n_hi = n_blk - blk_lo * nw
    cb = 16
    ec = cb * 128
    n_chunks = -(-(blk_lo + (1 if n_hi else 0)) // cb)
    assert blk_lo >= cb

    mesh = plsc.VectorSubcoreMesh(core_axis_name="c", subcore_axis_name="s",
                                  num_cores=nc, num_subcores=ns)

    @functools.partial(
        pl.kernel,
        out_type=jax.ShapeDtypeStruct((2, 2, n_edges), jnp.float32),
        mesh=mesh,
        compiler_params=pltpu.CompilerParams(needs_layout_passes=False),
        scratch_types=[
            pltpu.VMEM((n_tab_words,), jnp.float32),
            pltpu.VMEM((ec,), jnp.int32),
            pltpu.VMEM((ec,), jnp.int32),
            pltpu.VMEM((ec,), jnp.float32),
            pltpu.VMEM((ec,), jnp.float32),
            pltpu.VMEM((ec,), jnp.float32),
            pltpu.VMEM((ec,), jnp.float32),
        ],
    )
    def sc_edge_kernel(tab_hbm, src_hbm, dst_hbm, out_hbm,
                       tab_v, src_v, dst_v, cb0, cb1, cb2, cb3):
        w = lax.axis_index("s") * nc + lax.axis_index("c")
        pltpu.sync_copy(tab_hbm, tab_v)
        blk0 = w * blk_lo + jnp.minimum(w, n_hi)
        my_blks = blk_lo + jnp.where(w < n_hi, 1, 0)

        for k in range(n_chunks):
            e0 = (blk0 + jnp.minimum(k * cb, my_blks - cb)) * 128
            pltpu.sync_copy(src_hbm.at[pl.ds(e0, ec)], src_v)
            pltpu.sync_copy(dst_hbm.at[pl.ds(e0, ec)], dst_v)

            def group(g, carry):
                s = src_v[pl.ds(g * 16, 16)]
                d = dst_v[pl.ds(g * 16, 16)]
                s8 = s * 8
                d8 = d * 8 + 4
                for c, buf in ((0, cb0), (1, cb1), (2, cb2), (3, cb3)):
                    a = plsc.load_gather(tab_v, [s8 + c])
                    b = plsc.load_gather(tab_v, [d8 + c])
                    v = a + b
                    e = jnp.exp(-2.0 * jnp.abs(v))
                    r = (1.0 - e) / (1.0 + e)
                    r = jnp.where(v < 0.0, -r, r)
                    buf[pl.ds(g * 16, 16)] = r
                return carry

            lax.fori_loop(0, ec // 16, group, 0)
            for c, buf in ((0, cb0), (1, cb1), (2, cb2), (3, cb3)):
                pltpu.sync_copy(buf, out_hbm.at[c // 2, c % 2, pl.ds(e0, ec)])

    return sc_edge_kernel


def kernel(x, edge_index, edge_types, node_types, W):
    n, d = x.shape
    e = edge_index.shape[1]
    wcat = jnp.concatenate([W[:d], W[d:2 * d]], axis=1)
    tcat = jnp.concatenate([W[2 * d:2 * d + 4],
                            W[2 * d + 4:2 * d + 8]], axis=1)

    table = pl.pallas_call(
        _table_body,
        out_shape=jax.ShapeDtypeStruct((n, 8), jnp.float32),
    )(x, node_types.reshape(n, 1), wcat, tcat)

    tab_flat = table.reshape(-1)
    out = _make_sc_edge_kernel(tab_flat.shape[0], e)(
        tab_flat, edge_index[0], edge_index[1])
    return jnp.transpose(out, (2, 0, 1))


# speedup vs baseline: 33.9850x; 1.4403x over previous
import functools

import jax
import jax.numpy as jnp
from jax import lax
from jax.experimental import pallas as pl
from jax.experimental.pallas import tpu as pltpu
from jax.experimental.pallas import tpu_sc as plsc


def _table_body(x_ref, nt_ref, wcat_ref, tcat_ref, out_ref):
    acc = jnp.dot(x_ref[...], wcat_ref[...],
                  preferred_element_type=jnp.float32,
                  precision=lax.Precision.HIGHEST)
    nt = nt_ref[...]
    for t in range(4):
        mask = jnp.where(nt == t, 1.0, 0.0)
        acc = acc + mask * tcat_ref[t:t + 1, :]
    out_ref[...] = acc


def _make_sc_edge_kernel(n_tab_words, n_edges):
    nc, ns = 2, 16
    nw = nc * ns
    assert n_edges % 128 == 0
    n_blk = n_edges // 128
    blk_lo = n_blk // nw
    n_hi = n_blk - blk_lo * nw
    cb = 16
    ec = cb * 128
    n_chunks = -(-(blk_lo + (1 if n_hi else 0)) // cb)
    assert blk_lo >= cb

    mesh = plsc.VectorSubcoreMesh(core_axis_name="c", subcore_axis_name="s",
                                  num_cores=nc, num_subcores=ns)

    @functools.partial(
        pl.kernel,
        out_type=jax.ShapeDtypeStruct((2, 2, n_edges), jnp.float32),
        mesh=mesh,
        compiler_params=pltpu.CompilerParams(needs_layout_passes=False),
        scratch_types=[
            pltpu.VMEM((n_tab_words,), jnp.float32),
            pltpu.VMEM((ec,), jnp.int32),
            pltpu.VMEM((ec,), jnp.int32),
            pltpu.VMEM((ec,), jnp.float32),
            pltpu.VMEM((ec,), jnp.float32),
            pltpu.VMEM((ec,), jnp.float32),
            pltpu.VMEM((ec,), jnp.float32),
        ],
    )
    def sc_edge_kernel(tab_hbm, src_hbm, dst_hbm, out_hbm,
                       tab_v, src_v, dst_v, cb0, cb1, cb2, cb3):
        w = lax.axis_index("s") * nc + lax.axis_index("c")
        pltpu.sync_copy(tab_hbm, tab_v)
        blk0 = w * blk_lo + jnp.minimum(w, n_hi)
        my_blks = blk_lo + jnp.where(w < n_hi, 1, 0)

        for k in range(n_chunks):
            e0 = (blk0 + jnp.minimum(k * cb, my_blks - cb)) * 128
            pltpu.sync_copy(src_hbm.at[pl.ds(e0, ec)], src_v)
            pltpu.sync_copy(dst_hbm.at[pl.ds(e0, ec)], dst_v)

            @plsc.parallel_loop(0, ec // 16, unroll=8)
            def group(g):
                s = src_v[pl.ds(g * 16, 16)]
                d = dst_v[pl.ds(g * 16, 16)]
                s8 = s * 8
                d8 = d * 8 + 4
                for c, buf in ((0, cb0), (1, cb1), (2, cb2), (3, cb3)):
                    a = plsc.load_gather(tab_v, [s8 + c])
                    b = plsc.load_gather(tab_v, [d8 + c])
                    v = a + b
                    t = jnp.exp(jnp.minimum(v + v, 60.0))
                    buf[pl.ds(g * 16, 16)] = (t - 1.0) / (t + 1.0)
            for c, buf in ((0, cb0), (1, cb1), (2, cb2), (3, cb3)):
                pltpu.sync_copy(buf, out_hbm.at[c // 2, c % 2, pl.ds(e0, ec)])

    return sc_edge_kernel


def kernel(x, edge_index, edge_types, node_types, W):
    n, d = x.shape
    e = edge_index.shape[1]
    wcat = jnp.concatenate([W[:d], W[d:2 * d]], axis=1)
    tcat = jnp.concatenate([W[2 * d:2 * d + 4],
                            W[2 * d + 4:2 * d + 8]], axis=1)

    table = pl.pallas_call(
        _table_body,
        out_shape=jax.ShapeDtypeStruct((n, 8), jnp.float32),
    )(x, node_types.reshape(n, 1), wcat, tcat)

    tab_flat = table.reshape(-1)
    out = _make_sc_edge_kernel(tab_flat.shape[0], e)(
        tab_flat, edge_index[0], edge_index[1])
    return jnp.transpose(out, (2, 0, 1))


# speedup vs baseline: 39.3377x; 1.1575x over previous
import functools

import jax
import jax.numpy as jnp
from jax import lax
from jax.experimental import pallas as pl
from jax.experimental.pallas import tpu as pltpu
from jax.experimental.pallas import tpu_sc as plsc


def _table_body(x_ref, nt_ref, wcat_ref, tcat_ref, out_ref):
    acc = jnp.dot(x_ref[...], wcat_ref[...],
                  preferred_element_type=jnp.float32,
                  precision=lax.Precision.HIGHEST)
    nt = nt_ref[...]
    for t in range(4):
        mask = jnp.where(nt == t, 1.0, 0.0)
        acc = acc + mask * tcat_ref[t:t + 1, :]
    out_ref[...] = acc


def _make_sc_edge_kernel(n_tab_words, n_edges):
    nc, ns = 2, 16
    nw = nc * ns
    assert n_edges % 128 == 0
    n_blk = n_edges // 128
    blk_lo = n_blk // nw
    n_hi = n_blk - blk_lo * nw
    cb = 16
    ec = cb * 128
    n_chunks = -(-(blk_lo + (1 if n_hi else 0)) // cb)
    assert blk_lo >= cb

    mesh = plsc.VectorSubcoreMesh(core_axis_name="c", subcore_axis_name="s",
                                  num_cores=nc, num_subcores=ns)

    @functools.partial(
        pl.kernel,
        out_type=jax.ShapeDtypeStruct((2, 2, n_edges), jnp.float32),
        mesh=mesh,
        compiler_params=pltpu.CompilerParams(needs_layout_passes=False),
        scratch_types=[
            pltpu.VMEM((n_tab_words,), jnp.float32),
            pltpu.VMEM((ec,), jnp.int32),
            pltpu.VMEM((ec,), jnp.int32),
            pltpu.VMEM((ec,), jnp.float32),
            pltpu.VMEM((ec,), jnp.float32),
            pltpu.VMEM((ec,), jnp.float32),
            pltpu.VMEM((ec,), jnp.float32),
        ],
    )
    def sc_edge_kernel(tab_hbm, ei_hbm, out_hbm,
                       tab_v, src_v, dst_v, cb0, cb1, cb2, cb3):
        w = lax.axis_index("s") * nc + lax.axis_index("c")
        pltpu.sync_copy(tab_hbm, tab_v)
        blk0 = w * blk_lo + jnp.minimum(w, n_hi)
        my_blks = blk_lo + jnp.where(w < n_hi, 1, 0)

        for k in range(n_chunks):
            e0 = (blk0 + jnp.minimum(k * cb, my_blks - cb)) * 128
            pltpu.sync_copy(ei_hbm.at[0, pl.ds(e0, ec)], src_v)
            pltpu.sync_copy(ei_hbm.at[1, pl.ds(e0, ec)], dst_v)

            @plsc.parallel_loop(0, ec // 16, unroll=8)
            def group(g):
                s = src_v[pl.ds(g * 16, 16)]
                d = dst_v[pl.ds(g * 16, 16)]
                s8 = s * 8
                d8 = d * 8 + 4
                for c, buf in ((0, cb0), (1, cb1), (2, cb2), (3, cb3)):
                    a = plsc.load_gather(tab_v, [s8 + c])
                    b = plsc.load_gather(tab_v, [d8 + c])
                    v = a + b
                    t = jnp.exp(jnp.minimum(v + v, 60.0))
                    buf[pl.ds(g * 16, 16)] = (t - 1.0) / (t + 1.0)
            for c, buf in ((0, cb0), (1, cb1), (2, cb2), (3, cb3)):
                pltpu.sync_copy(buf, out_hbm.at[c // 2, c % 2, pl.ds(e0, ec)])

    return sc_edge_kernel


def kernel(x, edge_index, edge_types, node_types, W):
    n, d = x.shape
    e = edge_index.shape[1]
    wcat = jnp.concatenate([W[:d], W[d:2 * d]], axis=1)
    tcat = jnp.concatenate([W[2 * d:2 * d + 4],
                            W[2 * d + 4:2 * d + 8]], axis=1)

    table = pl.pallas_call(
        _table_body,
        out_shape=jax.ShapeDtypeStruct((n, 8), jnp.float32),
    )(x, node_types.reshape(n, 1), wcat, tcat)

    tab_flat = table.reshape(-1)
    out = _make_sc_edge_kernel(tab_flat.shape[0], e)(tab_flat, edge_index)
    return jnp.transpose(out, (2, 0, 1))
